# baseline (speedup 1.0000x reference)
import functools

import jax
import jax.numpy as jnp
from jax.experimental import pallas as pl
from jax.experimental.pallas import tpu as pltpu

N = 15000
E = 320000
EE = 160000
D = 128
H = 256

_BN_SCALE = 1.0 / (1.0 + 1e-5) ** 0.5

_ROWS = 1000
_EROWS = 4000


def _edge_emb_body(ea_ref, w_ref, b_ref, o_ref):
    o_ref[...] = (
        jnp.dot(ea_ref[...], w_ref[...], preferred_element_type=jnp.float32)
        + b_ref[...]
    )


def _edge_emb(edge_attr, W, b):
    e, de = edge_attr.shape
    d = W.shape[1]
    grid = e // _EROWS
    return pl.pallas_call(
        _edge_emb_body,
        grid=(grid,),
        in_specs=[
            pl.BlockSpec((_EROWS, de), lambda i: (i, 0)),
            pl.BlockSpec((de, d), lambda i: (0, 0)),
            pl.BlockSpec((1, d), lambda i: (0, 0)),
        ],
        out_specs=pl.BlockSpec((_EROWS, d), lambda i: (i, 0)),
        out_shape=jax.ShapeDtypeStruct((e, d), jnp.float32),
    )(edge_attr, W, b.reshape(1, d))


def _mlp_body(n_parts, relu_out, use_mask, *refs):
    h_ref = refs[0]
    parts = refs[1 : 1 + n_parts]
    eps_ref, w1_ref, b1_ref, w2_ref, b2_ref, g_ref, b_ref = refs[
        1 + n_parts : 8 + n_parts
    ]
    o_ref = refs[-1]
    z = h_ref[...] * (1.0 + eps_ref[0, 0])
    for p in parts:
        z = z + p[...]
    t = jnp.maximum(
        jnp.dot(z, w1_ref[...], preferred_element_type=jnp.float32) + b1_ref[...],
        0.0,
    )
    o = jnp.dot(t, w2_ref[...], preferred_element_type=jnp.float32) + b2_ref[...]
    o = o * (g_ref[...] * _BN_SCALE) + b_ref[...]
    if relu_out:
        o = jnp.maximum(o, 0.0)
    if use_mask:
        o = o * refs[8 + n_parts][...]
    o_ref[...] = o


def _gin_mlp(h, parts, eps, W1, b1, W2, b2, g, b, relu_out, mask=None):
    n, d = h.shape
    hh = W1.shape[1]
    grid = n // _ROWS
    n_parts = len(parts)
    blk = pl.BlockSpec((_ROWS, d), lambda i: (i, 0))
    in_specs = [blk]
    in_specs += [blk] * n_parts
    in_specs += [
        pl.BlockSpec((1, 1), lambda i: (0, 0)),
        pl.BlockSpec((d, hh), lambda i: (0, 0)),
        pl.BlockSpec((1, hh), lambda i: (0, 0)),
        pl.BlockSpec((hh, d), lambda i: (0, 0)),
        pl.BlockSpec((1, d), lambda i: (0, 0)),
        pl.BlockSpec((1, d), lambda i: (0, 0)),
        pl.BlockSpec((1, d), lambda i: (0, 0)),
    ]
    args = [h, *parts, eps.reshape(1, 1), W1, b1.reshape(1, hh), W2,
            b2.reshape(1, d), g.reshape(1, d), b.reshape(1, d)]
    if mask is not None:
        in_specs.append(pl.BlockSpec((_ROWS, 1), lambda i: (i, 0)))
        args.append(mask)
    return pl.pallas_call(
        functools.partial(_mlp_body, n_parts, relu_out, mask is not None),
        grid=(grid,),
        in_specs=in_specs,
        out_specs=blk,
        out_shape=jax.ShapeDtypeStruct((n, d), jnp.float32),
    )(*args)


def _combine_body(h_ref, p0_ref, o_ref, r_ref):
    s = h_ref[...] + p0_ref[...]
    o_ref[...] = s
    r_ref[...] = jnp.maximum(s, 0.0)


def _combine(h, p0):
    n, d = h.shape
    blk = pl.BlockSpec((_ROWS, d), lambda i: (i, 0))
    return pl.pallas_call(
        _combine_body,
        grid=(n // _ROWS,),
        in_specs=[blk, blk],
        out_specs=[blk, blk],
        out_shape=[
            jax.ShapeDtypeStruct((n, d), jnp.float32),
            jax.ShapeDtypeStruct((n, d), jnp.float32),
        ],
    )(h, p0)


def _init_body(m_ref, e_ref, o_ref):
    o_ref[...] = m_ref[...] * e_ref[...]


def _init_h(emb, mask):
    n = mask.shape[0]
    d = emb.shape[1]
    return pl.pallas_call(
        _init_body,
        grid=(n // _ROWS,),
        in_specs=[
            pl.BlockSpec((_ROWS, 1), lambda i: (i, 0)),
            pl.BlockSpec((1, d), lambda i: (0, 0)),
        ],
        out_specs=pl.BlockSpec((_ROWS, d), lambda i: (i, 0)),
        out_shape=jax.ShapeDtypeStruct((n, d), jnp.float32),
    )(mask, emb)


def _segsum_main(h, e_emb, src, dst):
    msg = jnp.maximum(h[src] + e_emb, 0.0)
    return jax.ops.segment_sum(msg, dst, num_segments=h.shape[0])


def _segsum_plain(v, gidx, sidx):
    return jax.ops.segment_sum(v[gidx], sidx, num_segments=v.shape[0])


def kernel(x, edge_index, edge_attr, batch, expander_edge_index,
           expander_node_mask, num_nodes, emb, Wedge, bedge, eps_main,
           W1, b1, W2, b2, bn_g, bn_b, eps_r, rW1, rb1, rW2, rb2,
           rbn_g, rbn_b):
    mask = expander_node_mask[:, None]
    h = _init_h(emb, mask)
    src, dst = edge_index[0], edge_index[1]
    esrc, edst = expander_edge_index[0], expander_edge_index[1]
    L = W1.shape[0]
    for layer in range(L):
        e_emb = _edge_emb(edge_attr, Wedge[layer], bedge[layer])
        aggr = _segsum_main(h, e_emb, src, dst)
        last = layer == L - 1
        h = _gin_mlp(h, [aggr], eps_main[layer], W1[layer], b1[layer],
                     W2[layer], b2[layer], bn_g[layer], bn_b[layer],
                     relu_out=not last, mask=None if last else mask)
        if not last:
            h_edge = _segsum_plain(h, esrc, edst)
            h_new, r = _combine(h, h_edge)
            aggr_r = _segsum_plain(r, edst, esrc)
            h = _gin_mlp(h_new, [aggr_r], eps_r[layer], rW1[layer],
                         rb1[layer], rW2[layer], rb2[layer], rbn_g[layer],
                         rbn_b[layer], relu_out=True)
    return h


# speedup vs baseline: 1.5649x; 1.5649x over previous
import functools

import jax
import jax.numpy as jnp
from jax import lax
from jax.experimental import pallas as pl
from jax.experimental.pallas import tpu as pltpu
from jax.experimental.pallas import tpu_sc as plsc

N = 15000
D = 128
H = 256

_BN_SCALE = 1.0 / (1.0 + 1e-5) ** 0.5

_ROWS = 120
_EROWS = 4000

_C = 112
_NT = 16
_HALF = 7680
_ACC_ROWS = 7800
_DUMMY = 7700
_HBND = _HALF // _ROWS
_ABND = _ACC_ROWS // _ROWS
_ZR = 488


def _edge_emb_body(ea_ref, w_ref, b_ref, o_ref):
    o_ref[...] = (
        jnp.dot(ea_ref[...], w_ref[...], preferred_element_type=jnp.float32)
        + b_ref[...]
    )


def _edge_emb(edge_attr, W, b, e_pad):
    e, de = edge_attr.shape
    d = W.shape[1]
    return pl.pallas_call(
        _edge_emb_body,
        grid=(e // _EROWS,),
        in_specs=[
            pl.BlockSpec((_EROWS, de), lambda i: (i, 0)),
            pl.BlockSpec((de, d), lambda i: (0, 0)),
            pl.BlockSpec((1, d), lambda i: (0, 0)),
        ],
        out_specs=pl.BlockSpec((_EROWS, d), lambda i: (i, 0)),
        out_shape=jax.ShapeDtypeStruct((e_pad, d), jnp.float32),
    )(edge_attr, W, b.reshape(1, d))


def _agg_spec():
    gap = _ABND - _HBND
    return pl.BlockSpec(
        (_ROWS, D),
        lambda i: (i + gap * (i >= _HBND).astype(jnp.int32), 0),
    )


def _mlp_body(relu_out, use_mask, *refs):
    h_ref, agg_ref = refs[0], refs[1]
    eps_ref, w1_ref, b1_ref, w2_ref, b2_ref, g_ref, b_ref = refs[2:9]
    o_ref = refs[-1]
    z = h_ref[...] * (1.0 + eps_ref[0, 0]) + agg_ref[...]
    t = jnp.maximum(
        jnp.dot(z, w1_ref[...], preferred_element_type=jnp.float32) + b1_ref[...],
        0.0,
    )
    o = jnp.dot(t, w2_ref[...], preferred_element_type=jnp.float32) + b2_ref[...]
    o = o * (g_ref[...] * _BN_SCALE) + b_ref[...]
    if relu_out:
        o = jnp.maximum(o, 0.0)
    if use_mask:
        o = o * refs[9][...]
    o_ref[...] = o


def _gin_mlp(h, agg, eps, W1, b1, W2, b2, g, b, relu_out, mask=None):
    d = W1.shape[0]
    hh = W1.shape[1]
    blk = pl.BlockSpec((_ROWS, d), lambda i: (i, 0))
    in_specs = [
        blk,
        _agg_spec(),
        pl.BlockSpec((1, 1), lambda i: (0, 0)),
        pl.BlockSpec((d, hh), lambda i: (0, 0)),
        pl.BlockSpec((1, hh), lambda i: (0, 0)),
        pl.BlockSpec((hh, d), lambda i: (0, 0)),
        pl.BlockSpec((1, d), lambda i: (0, 0)),
        pl.BlockSpec((1, d), lambda i: (0, 0)),
        pl.BlockSpec((1, d), lambda i: (0, 0)),
    ]
    args = [h, agg, eps.reshape(1, 1), W1, b1.reshape(1, hh), W2,
            b2.reshape(1, d), g.reshape(1, d), b.reshape(1, d)]
    if mask is not None:
        in_specs.append(pl.BlockSpec((_ROWS, 1), lambda i: (i, 0)))
        args.append(mask)
    return pl.pallas_call(
        functools.partial(_mlp_body, relu_out, mask is not None),
        grid=(N // _ROWS,),
        in_specs=in_specs,
        out_specs=blk,
        out_shape=jax.ShapeDtypeStruct((N, D), jnp.float32),
    )(*args)


def _combine_body(h_ref, agg_ref, o_ref, r_ref):
    s = h_ref[...] + agg_ref[...]
    o_ref[...] = s
    r_ref[...] = jnp.maximum(s, 0.0)


def _combine(h, agg):
    blk = pl.BlockSpec((_ROWS, D), lambda i: (i, 0))
    sh = jax.ShapeDtypeStruct((N, D), jnp.float32)
    return pl.pallas_call(
        _combine_body,
        grid=(N // _ROWS,),
        in_specs=[blk, _agg_spec()],
        out_specs=[blk, blk],
        out_shape=[sh, sh],
    )(h, agg)


def _init_body(m_ref, e_ref, o_ref):
    o_ref[...] = m_ref[...] * e_ref[...]


def _init_h(emb, mask):
    d = emb.shape[1]
    return pl.pallas_call(
        _init_body,
        grid=(N // _ROWS,),
        in_specs=[
            pl.BlockSpec((_ROWS, 1), lambda i: (i, 0)),
            pl.BlockSpec((1, d), lambda i: (0, 0)),
        ],
        out_specs=pl.BlockSpec((_ROWS, d), lambda i: (i, 0)),
        out_shape=jax.ShapeDtypeStruct((N, d), jnp.float32),
    )(mask, emb)


def _route(c, sbuf):
    base = c * _HALF
    for k in range(_C // 16):
        sl = pl.ds(k * 16, 16)
        v = sbuf[sl]
        t = v - base
        own = (t >= 0) & (t < _HALF)
        sbuf[sl] = jnp.where(own, t, _DUMMY)


def _sc_body(nchunks, fused, vals, gidx, sidx, *refs):
    if fused:
        lin, zeros, out, acc, gbuf, sbuf, vbuf, hbuf = refs
    else:
        zeros, out, acc, gbuf, sbuf, vbuf = refs
        lin = hbuf = None
    c = lax.axis_index("c")
    s = lax.axis_index("s")

    zbase = pl.multiple_of(jnp.minimum(s * _ZR, _ACC_ROWS - _ZR), 8)
    pltpu.sync_copy(zeros, acc.at[pl.ds(zbase, _ZR), :])
    plsc.subcore_barrier()

    wbase = s * (nchunks * _C)

    def chunk(i, carry):
        ebase = pl.multiple_of(wbase + i * _C, 8)
        pltpu.sync_copy(gidx.at[pl.ds(ebase, _C)], gbuf)
        pltpu.sync_copy(sidx.at[pl.ds(ebase, _C)], sbuf)
        if fused:
            pltpu.sync_copy(lin.at[pl.ds(ebase, _C), :], vbuf)
            pltpu.sync_copy(vals.at[gbuf], hbuf)
            _route(c, sbuf)

            def row(r, carry2):
                for k in range(D // 16):
                    sl = pl.ds(k * 16, 16)
                    vbuf[r, sl] = jnp.maximum(hbuf[r, sl] + vbuf[r, sl], 0.0)
                return carry2

            lax.fori_loop(0, _C, row, None)
        else:
            pltpu.sync_copy(vals.at[gbuf], vbuf)
            _route(c, sbuf)
        pltpu.sync_copy(vbuf, acc.at[sbuf], add=True)
        return carry

    lax.fori_loop(0, nchunks, chunk, None)
    plsc.subcore_barrier()

    rbase = pl.multiple_of(jnp.minimum(s * _ZR, _ACC_ROWS - _ZR), 8)
    pltpu.sync_copy(acc.at[pl.ds(rbase, _ZR), :],
                    out.at[pl.ds(c * _ACC_ROWS + rbase, _ZR), :])


def _sc_segsum(vals, gidx, sidx, lin=None):
    e_pad = gidx.shape[0]
    nchunks = e_pad // (_NT * _C)
    zeros = jnp.zeros((_ZR, D), jnp.float32)
    mesh = plsc.VectorSubcoreMesh(core_axis_name="c", subcore_axis_name="s")
    scratch = [
        pltpu.VMEM_SHARED((_ACC_ROWS, D), jnp.float32),
        pltpu.VMEM((_C,), jnp.int32),
        pltpu.VMEM((_C,), jnp.int32),
        pltpu.VMEM((_C, D), jnp.float32),
    ]
    out_type = jax.ShapeDtypeStruct((2 * _ACC_ROWS, D), jnp.float32)
    if lin is not None:
        scratch.append(pltpu.VMEM((_C, D), jnp.float32))
        body = functools.partial(_sc_body, nchunks, True)
        return pl.kernel(body, out_type=out_type, mesh=mesh,
                         scratch_types=scratch)(vals, gidx, sidx, lin, zeros)
    body = functools.partial(_sc_body, nchunks, False)
    return pl.kernel(body, out_type=out_type, mesh=mesh,
                     scratch_types=scratch)(vals, gidx, sidx, zeros)


def _pad_idx(idx, e_pad, fill):
    e = idx.shape[0]
    return jnp.concatenate(
        [idx, jnp.full((e_pad - e,), fill, jnp.int32)])


def kernel(x, edge_index, edge_attr, batch, expander_edge_index,
           expander_node_mask, num_nodes, emb, Wedge, bedge, eps_main,
           W1, b1, W2, b2, bn_g, bn_b, eps_r, rW1, rb1, rW2, rb2,
           rbn_g, rbn_b):
    mask = expander_node_mask[:, None]
    h = _init_h(emb, mask)

    e = edge_index.shape[1]
    ee = expander_edge_index.shape[1]
    grp = _NT * _C
    e_pad = ((e + grp - 1) // grp) * grp
    ee_pad = ((ee + grp - 1) // grp) * grp
    src = _pad_idx(edge_index[0], e_pad, 0)
    dst = _pad_idx(edge_index[1], e_pad, N)
    esrc = _pad_idx(expander_edge_index[0], ee_pad, 0)
    edst = _pad_idx(expander_edge_index[1], ee_pad, N)

    L = W1.shape[0]
    for layer in range(L):
        e_emb = _edge_emb(edge_attr, Wedge[layer], bedge[layer], e_pad)
        agg = _sc_segsum(h, src, dst, lin=e_emb)
        last = layer == L - 1
        h = _gin_mlp(h, agg, eps_main[layer], W1[layer], b1[layer],
                     W2[layer], b2[layer], bn_g[layer], bn_b[layer],
                     relu_out=not last, mask=None if last else mask)
        if not last:
            agg = _sc_segsum(h, esrc, edst)
            h, r = _combine(h, agg)
            agg = _sc_segsum(r, edst, esrc)
            h = _gin_mlp(h, agg, eps_r[layer], rW1[layer], rb1[layer],
                         rW2[layer], rb2[layer], rbn_g[layer], rbn_b[layer],
                         relu_out=True)
    return h


# speedup vs baseline: 2.1396x; 1.3672x over previous
import functools

import jax
import jax.numpy as jnp
from jax import lax
from jax.experimental import pallas as pl
from jax.experimental.pallas import tpu as pltpu
from jax.experimental.pallas import tpu_sc as plsc

N = 15000
D = 128
H = 256

_BN_SCALE = 1.0 / (1.0 + 1e-5) ** 0.5

_ROWS = 120
_EROWS = 4000

_C = 112
_NT = 16
_HALF = 7680
_ACC_ROWS = 7800
_DUMMY = 7700
_HBND = _HALF // _ROWS
_ABND = _ACC_ROWS // _ROWS
_ZR = 488


def _edge_emb_body(ea_ref, w_ref, b_ref, o_ref):
    o_ref[...] = (
        jnp.dot(ea_ref[...], w_ref[...], preferred_element_type=jnp.float32)
        + b_ref[...]
    )


def _edge_emb(edge_attr, W, b, e_pad):
    e, de = edge_attr.shape
    d = W.shape[1]
    return pl.pallas_call(
        _edge_emb_body,
        grid=(e // _EROWS,),
        in_specs=[
            pl.BlockSpec((_EROWS, de), lambda i: (i, 0)),
            pl.BlockSpec((de, d), lambda i: (0, 0)),
            pl.BlockSpec((1, d), lambda i: (0, 0)),
        ],
        out_specs=pl.BlockSpec((_EROWS, d), lambda i: (i, 0)),
        out_shape=jax.ShapeDtypeStruct((e_pad, d), jnp.float32),
    )(edge_attr, W, b.reshape(1, d))


def _agg_spec():
    gap = _ABND - _HBND
    return pl.BlockSpec(
        (_ROWS, D),
        lambda i: (i + gap * (i >= _HBND).astype(jnp.int32), 0),
    )


def _mlp_body(relu_out, use_mask, *refs):
    h_ref, agg_ref = refs[0], refs[1]
    eps_ref, w1_ref, b1_ref, w2_ref, b2_ref, g_ref, b_ref = refs[2:9]
    o_ref = refs[-1]
    z = h_ref[...] * (1.0 + eps_ref[0, 0]) + agg_ref[...]
    t = jnp.maximum(
        jnp.dot(z, w1_ref[...], preferred_element_type=jnp.float32) + b1_ref[...],
        0.0,
    )
    o = jnp.dot(t, w2_ref[...], preferred_element_type=jnp.float32) + b2_ref[...]
    o = o * (g_ref[...] * _BN_SCALE) + b_ref[...]
    if relu_out:
        o = jnp.maximum(o, 0.0)
    if use_mask:
        o = o * refs[9][...]
    o_ref[...] = o


def _gin_mlp(h, agg, eps, W1, b1, W2, b2, g, b, relu_out, mask=None):
    d = W1.shape[0]
    hh = W1.shape[1]
    blk = pl.BlockSpec((_ROWS, d), lambda i: (i, 0))
    in_specs = [
        blk,
        _agg_spec(),
        pl.BlockSpec((1, 1), lambda i: (0, 0)),
        pl.BlockSpec((d, hh), lambda i: (0, 0)),
        pl.BlockSpec((1, hh), lambda i: (0, 0)),
        pl.BlockSpec((hh, d), lambda i: (0, 0)),
        pl.BlockSpec((1, d), lambda i: (0, 0)),
        pl.BlockSpec((1, d), lambda i: (0, 0)),
        pl.BlockSpec((1, d), lambda i: (0, 0)),
    ]
    args = [h, agg, eps.reshape(1, 1), W1, b1.reshape(1, hh), W2,
            b2.reshape(1, d), g.reshape(1, d), b.reshape(1, d)]
    if mask is not None:
        in_specs.append(pl.BlockSpec((_ROWS, 1), lambda i: (i, 0)))
        args.append(mask)
    return pl.pallas_call(
        functools.partial(_mlp_body, relu_out, mask is not None),
        grid=(N // _ROWS,),
        in_specs=in_specs,
        out_specs=blk,
        out_shape=jax.ShapeDtypeStruct((N, D), jnp.float32),
    )(*args)


def _combine_body(h_ref, agg_ref, o_ref, r_ref):
    s = h_ref[...] + agg_ref[...]
    o_ref[...] = s
    r_ref[...] = jnp.maximum(s, 0.0)


def _combine(h, agg):
    blk = pl.BlockSpec((_ROWS, D), lambda i: (i, 0))
    sh = jax.ShapeDtypeStruct((N, D), jnp.float32)
    return pl.pallas_call(
        _combine_body,
        grid=(N // _ROWS,),
        in_specs=[blk, _agg_spec()],
        out_specs=[blk, blk],
        out_shape=[sh, sh],
    )(h, agg)


def _init_body(m_ref, e_ref, o_ref):
    o_ref[...] = m_ref[...] * e_ref[...]


def _init_h(emb, mask):
    d = emb.shape[1]
    return pl.pallas_call(
        _init_body,
        grid=(N // _ROWS,),
        in_specs=[
            pl.BlockSpec((_ROWS, 1), lambda i: (i, 0)),
            pl.BlockSpec((1, d), lambda i: (0, 0)),
        ],
        out_specs=pl.BlockSpec((_ROWS, d), lambda i: (i, 0)),
        out_shape=jax.ShapeDtypeStruct((N, d), jnp.float32),
    )(mask, emb)


def _route(c, sbuf):
    base = c * _HALF
    for k in range(_C // 16):
        sl = pl.ds(k * 16, 16)
        v = sbuf[sl]
        t = v - base
        own = (t >= 0) & (t < _HALF)
        sbuf[sl] = jnp.where(own, t, _DUMMY)


def _sc_body(nchunks, fused, vals, gidx, sidx, *refs):
    if fused:
        (lin, zeros, out, acc,
         g0, s0, v0, h0, g1, s1, v1, h1, si0, si1, sd0, sd1) = refs
        slot0 = (g0, s0, v0, h0, si0, sd0)
        slot1 = (g1, s1, v1, h1, si1, sd1)
    else:
        (zeros, out, acc,
         g0, s0, v0, g1, s1, v1, si0, si1, sd0, sd1) = refs
        lin = None
        slot0 = (g0, s0, v0, None, si0, sd0)
        slot1 = (g1, s1, v1, None, si1, sd1)
    c = lax.axis_index("c")
    s = lax.axis_index("s")

    zbase = pl.multiple_of(jnp.minimum(s * _ZR, _ACC_ROWS - _ZR), 8)
    pltpu.sync_copy(zeros, acc.at[pl.ds(zbase, _ZR), :])
    plsc.subcore_barrier()

    wbase = s * (nchunks * _C)

    def ebase(i):
        return pl.multiple_of(wbase + i * _C, 8)

    def issue_idx(i, sl):
        g, sb, _, _, si, _ = sl
        pltpu.async_copy(gidx.at[pl.ds(ebase(i), _C)], g, si)
        pltpu.async_copy(sidx.at[pl.ds(ebase(i), _C)], sb, si)

    def wait_idx(sl):
        g, sb, _, _, si, _ = sl
        pltpu.make_async_copy(gidx.at[pl.ds(0, _C)], g, si).wait()
        pltpu.make_async_copy(sidx.at[pl.ds(0, _C)], sb, si).wait()

    def issue_data(i, sl):
        g, _, v, h, _, sd = sl
        if fused:
            pltpu.async_copy(lin.at[pl.ds(ebase(i), _C), :], v, sd)
            pltpu.async_copy(vals.at[g], h, sd)
        else:
            pltpu.async_copy(vals.at[g], v, sd)

    def wait_data(sl):
        _, _, v, h, _, sd = sl
        if fused:
            pltpu.make_async_copy(lin.at[pl.ds(0, _C), :], v, sd).wait()
            pltpu.make_async_copy(vals.at[pl.ds(0, _C), :], h, sd).wait()
        else:
            pltpu.make_async_copy(vals.at[pl.ds(0, _C), :], v, sd).wait()

    def compute(sl):
        _, sb, v, h, _, _ = sl
        _route(c, sb)
        if fused:

            def row(r, carry2):
                for k in range(D // 16):
                    slc = pl.ds(k * 16, 16)
                    v[r, slc] = jnp.maximum(h[r, slc] + v[r, slc], 0.0)
                return carry2

            lax.fori_loop(0, _C, row, None)
        pltpu.sync_copy(v, acc.at[sb], add=True)

    issue_idx(0, slot0)
    issue_idx(1, slot1)
    wait_idx(slot0)
    issue_data(0, slot0)

    def pair(j, carry):
        a = 2 * j
        wait_data(slot0)
        wait_idx(slot1)
        issue_data(a + 1, slot1)
        compute(slot0)
        issue_idx(a + 2, slot0)
        wait_data(slot1)
        wait_idx(slot0)
        issue_data(a + 2, slot0)
        compute(slot1)
        issue_idx(a + 3, slot1)
        return carry

    lax.fori_loop(0, nchunks // 2 - 1, pair, None)

    wait_data(slot0)
    wait_idx(slot1)
    issue_data(nchunks - 1, slot1)
    compute(slot0)
    wait_data(slot1)
    compute(slot1)
    plsc.subcore_barrier()

    rbase = pl.multiple_of(jnp.minimum(s * _ZR, _ACC_ROWS - _ZR), 8)
    pltpu.sync_copy(acc.at[pl.ds(rbase, _ZR), :],
                    out.at[pl.ds(c * _ACC_ROWS + rbase, _ZR), :])


def _sc_segsum(vals, gidx, sidx, lin=None):
    e_pad = gidx.shape[0]
    nchunks = e_pad // (_NT * _C)
    zeros = jnp.zeros((_ZR, D), jnp.float32)
    mesh = plsc.VectorSubcoreMesh(core_axis_name="c", subcore_axis_name="s")
    ibuf = pltpu.VMEM((_C,), jnp.int32)
    dbuf = pltpu.VMEM((_C, D), jnp.float32)
    sem = pltpu.SemaphoreType.DMA
    acc_t = pltpu.VMEM_SHARED((_ACC_ROWS, D), jnp.float32)
    out_type = jax.ShapeDtypeStruct((2 * _ACC_ROWS, D), jnp.float32)
    if lin is not None:
        scratch = [acc_t, ibuf, ibuf, dbuf, dbuf, ibuf, ibuf, dbuf, dbuf,
                   sem, sem, sem, sem]
        body = functools.partial(_sc_body, nchunks, True)
        return pl.kernel(body, out_type=out_type, mesh=mesh,
                         scratch_types=scratch)(vals, gidx, sidx, lin, zeros)
    scratch = [acc_t, ibuf, ibuf, dbuf, ibuf, ibuf, dbuf, sem, sem, sem, sem]
    body = functools.partial(_sc_body, nchunks, False)
    return pl.kernel(body, out_type=out_type, mesh=mesh,
                     scratch_types=scratch)(vals, gidx, sidx, zeros)


def _pad_idx(idx, e_pad, fill):
    e = idx.shape[0]
    return jnp.concatenate(
        [idx, jnp.full((e_pad - e,), fill, jnp.int32)])


def kernel(x, edge_index, edge_attr, batch, expander_edge_index,
           expander_node_mask, num_nodes, emb, Wedge, bedge, eps_main,
           W1, b1, W2, b2, bn_g, bn_b, eps_r, rW1, rb1, rW2, rb2,
           rbn_g, rbn_b):
    mask = expander_node_mask[:, None]
    h = _init_h(emb, mask)

    e = edge_index.shape[1]
    ee = expander_edge_index.shape[1]
    grp = 2 * _NT * _C
    e_pad = ((e + grp - 1) // grp) * grp
    ee_pad = ((ee + grp - 1) // grp) * grp
    src = _pad_idx(edge_index[0], e_pad, 0)
    dst = _pad_idx(edge_index[1], e_pad, N)
    esrc = _pad_idx(expander_edge_index[0], ee_pad, 0)
    edst = _pad_idx(expander_edge_index[1], ee_pad, N)

    L = W1.shape[0]
    for layer in range(L):
        e_emb = _edge_emb(edge_attr, Wedge[layer], bedge[layer], e_pad)
        agg = _sc_segsum(h, src, dst, lin=e_emb)
        last = layer == L - 1
        h = _gin_mlp(h, agg, eps_main[layer], W1[layer], b1[layer],
                     W2[layer], b2[layer], bn_g[layer], bn_b[layer],
                     relu_out=not last, mask=None if last else mask)
        if not last:
            agg = _sc_segsum(h, esrc, edst)
            h, r = _combine(h, agg)
            agg = _sc_segsum(r, edst, esrc)
            h = _gin_mlp(h, agg, eps_r[layer], rW1[layer], rb1[layer],
                         rW2[layer], rb2[layer], rbn_g[layer], rbn_b[layer],
                         relu_out=True)
    return h
